# Initial kernel scaffold; baseline (speedup 1.0000x reference)
import jax
import jax.numpy as jnp
from jax.experimental import pallas as pl


def kernel(x, edge_index, W1, b1, W2, b2, W3, b3, Wf, bf):
    raise NotImplementedError("write your pallas kernel here")


# speedup vs baseline: 44.1269x; 44.1269x over previous
import functools

import jax
import jax.numpy as jnp
from jax import lax
from jax.experimental import pallas as pl
from jax.experimental.pallas import tpu as pltpu
from jax.experimental.pallas import tpu_sc as plsc

_N = 50000
_D = 128
_E = 1600000

_NC = 2
_NS = 16
_NW = _NC * _NS
_CH = 125
_EPW = _E // _NW
_NCH = _EPW // _CH
_INNER = 8
_OUTER = _NCH // _INNER

_RPT = 3128
_RPT_LAST = _N - (_NS - 1) * _RPT

_BLK = 1000
_G = _N // _BLK


def _sc_mesh():
  return plsc.VectorSubcoreMesh(core_axis_name="c", subcore_axis_name="s")


def _sc_degree(dst2, W=8):
  @functools.partial(
      pl.kernel,
      out_type=jax.ShapeDtypeStruct((_NC, _N, W), jnp.float32),
      mesh=_sc_mesh(),
      scratch_types=[
          pltpu.VMEM((_INNER, _CH), jnp.int32),
          pltpu.VMEM((_CH, W), jnp.float32),
          pltpu.VMEM_SHARED((_N, W), jnp.float32),
          pltpu.SemaphoreType.DMA,
      ],
      compiler_params=pltpu.CompilerParams(use_tc_tiling_on_sc=False),
  )
  def k(dst_h, ones_h, zeros_h, out_h, didx, obuf, acc, ssem):
    c = lax.axis_index("c")
    s = lax.axis_index("s")
    wid = c * _NS + s

    @pl.when(s == 0)
    def _():
      pltpu.sync_copy(zeros_h, acc)

    pltpu.sync_copy(ones_h, obuf)
    plsc.subcore_barrier()

    def outer(o, carry):
      r0 = wid * _NCH + o * _INNER
      pltpu.sync_copy(dst_h.at[pl.ds(r0, _INNER)], didx)
      cps = [
          pltpu.async_copy(obuf, acc.at[didx.at[i]], ssem, add=True)
          for i in range(_INNER)
      ]
      for cp in cps:
        cp.wait()
      return carry

    lax.fori_loop(0, _OUTER, outer, 0)
    plsc.subcore_barrier()

    @pl.when(s == 0)
    def _():
      pltpu.sync_copy(acc, out_h.at[c])

  ones = jnp.ones((_CH, W), jnp.float32)
  zeros = jnp.zeros((_N, W), jnp.float32)
  return k(dst2, ones, zeros)


def _sc_scatter(g, src2, dst2, F):
  nbuf = 4 if F >= 32 else _INNER

  @functools.partial(
      pl.kernel,
      out_type=jax.ShapeDtypeStruct((_NC, _N, F), jnp.float32),
      mesh=_sc_mesh(),
      scratch_types=[
          pltpu.VMEM((_INNER, _CH), jnp.int32),
          pltpu.VMEM((_INNER, _CH), jnp.int32),
          pltpu.VMEM((nbuf, _CH, F), jnp.float32),
          pltpu.VMEM_SHARED((_N, F), jnp.float32),
          pltpu.SemaphoreType.DMA,
          pltpu.SemaphoreType.DMA,
      ],
      compiler_params=pltpu.CompilerParams(use_tc_tiling_on_sc=False),
  )
  def k(g_h, src_h, dst_h, zeros_h, out_h, sidx, didx, rows, acc, gsem, ssem):
    c = lax.axis_index("c")
    s = lax.axis_index("s")
    wid = c * _NS + s

    @pl.when(s < _NS - 1)
    def _():
      pltpu.sync_copy(zeros_h, acc.at[pl.ds(s * _RPT, _RPT)])

    @pl.when(s == _NS - 1)
    def _():
      pltpu.sync_copy(zeros_h.at[pl.ds(0, _RPT_LAST)],
                      acc.at[pl.ds((_NS - 1) * _RPT, _RPT_LAST)])

    plsc.subcore_barrier()

    def outer(o, carry):
      r0 = wid * _NCH + o * _INNER
      pltpu.sync_copy(src_h.at[pl.ds(r0, _INNER)], sidx)
      pltpu.sync_copy(dst_h.at[pl.ds(r0, _INNER)], didx)
      gcps = [None] * _INNER
      scps = [None] * _INNER
      for i in range(nbuf):
        gcps[i] = pltpu.async_copy(g_h.at[sidx.at[i]], rows.at[i % nbuf],
                                   gsem)
      for i in range(_INNER):
        gcps[i].wait()
        scps[i] = pltpu.async_copy(rows.at[i % nbuf], acc.at[didx.at[i]],
                                   ssem, add=True)
        nxt = i + nbuf
        if nxt < _INNER:
          scps[i].wait()
          gcps[nxt] = pltpu.async_copy(g_h.at[sidx.at[nxt]],
                                       rows.at[nxt % nbuf], gsem)
      for i in range(max(0, _INNER - nbuf), _INNER):
        scps[i].wait()
      return carry

    lax.fori_loop(0, _OUTER, outer, 0)
    plsc.subcore_barrier()

    @pl.when(s < _NS - 1)
    def _():
      pltpu.sync_copy(acc.at[pl.ds(s * _RPT, _RPT)],
                      out_h.at[c, pl.ds(s * _RPT, _RPT)])

    @pl.when(s == _NS - 1)
    def _():
      pltpu.sync_copy(acc.at[pl.ds((_NS - 1) * _RPT, _RPT_LAST)],
                      out_h.at[c, pl.ds((_NS - 1) * _RPT, _RPT_LAST)])

  zeros = jnp.zeros((_RPT, F), jnp.float32)
  return k(g, src2, dst2, zeros)


def _tc_first(x, W1, d0, d1):
  def body(x_r, w_r, d0_r, d1_r, dinv_r, h_r, g_r):
    deg = 1.0 + d0_r[...] + d1_r[...]
    dinv = lax.rsqrt(deg)
    h = jnp.dot(x_r[...], w_r[...], preferred_element_type=jnp.float32)
    dinv_r[...] = dinv
    h_r[...] = h
    g_r[...] = h * dinv

  F = W1.shape[1]
  return pl.pallas_call(
      body,
      grid=(_G,),
      in_specs=[
          pl.BlockSpec((_BLK, _D), lambda i: (i, 0)),
          pl.BlockSpec((_D, F), lambda i: (0, 0)),
          pl.BlockSpec((_BLK, 1), lambda i: (i, 0)),
          pl.BlockSpec((_BLK, 1), lambda i: (i, 0)),
      ],
      out_specs=[
          pl.BlockSpec((_BLK, 1), lambda i: (i, 0)),
          pl.BlockSpec((_BLK, F), lambda i: (i, 0)),
          pl.BlockSpec((_BLK, F), lambda i: (i, 0)),
      ],
      out_shape=[
          jax.ShapeDtypeStruct((_N, 1), jnp.float32),
          jax.ShapeDtypeStruct((_N, F), jnp.float32),
          jax.ShapeDtypeStruct((_N, F), jnp.float32),
      ],
  )(x, W1, d0, d1)


def _tc_mid(p0, p1, hlin, dinv, b, Wn):
  def body(p0_r, p1_r, h_r, di_r, b_r, w_r, h2_r, g2_r):
    dinv = di_r[...]
    h = jnp.maximum(
        dinv * (p0_r[...] + p1_r[...]) + (dinv * dinv) * h_r[...] + b_r[...],
        0.0)
    h2 = jnp.dot(h, w_r[...], preferred_element_type=jnp.float32)
    h2_r[...] = h2
    g2_r[...] = h2 * dinv

  Fin, Fout = Wn.shape
  return pl.pallas_call(
      body,
      grid=(_G,),
      in_specs=[
          pl.BlockSpec((_BLK, Fin), lambda i: (i, 0)),
          pl.BlockSpec((_BLK, Fin), lambda i: (i, 0)),
          pl.BlockSpec((_BLK, Fin), lambda i: (i, 0)),
          pl.BlockSpec((_BLK, 1), lambda i: (i, 0)),
          pl.BlockSpec((1, Fin), lambda i: (0, 0)),
          pl.BlockSpec((Fin, Fout), lambda i: (0, 0)),
      ],
      out_specs=[
          pl.BlockSpec((_BLK, Fout), lambda i: (i, 0)),
          pl.BlockSpec((_BLK, Fout), lambda i: (i, 0)),
      ],
      out_shape=[
          jax.ShapeDtypeStruct((_N, Fout), jnp.float32),
          jax.ShapeDtypeStruct((_N, Fout), jnp.float32),
      ],
  )(p0, p1, hlin, dinv, b, Wn)


def _tc_final(p0, p1, hlin, dinv, b, Wf, bf):
  def body(p0_r, p1_r, h_r, di_r, b_r, wf_r, bf_r, o_r):
    dinv = di_r[...]
    h = jnp.maximum(
        dinv * (p0_r[...] + p1_r[...]) + (dinv * dinv) * h_r[...] + b_r[...],
        0.0)
    logits = jnp.dot(h, wf_r[...], preferred_element_type=jnp.float32)
    logits = logits + bf_r[...]
    m = jnp.max(logits, axis=1, keepdims=True)
    ex = jnp.exp(logits - m)
    o_r[...] = logits - m - jnp.log(jnp.sum(ex, axis=1, keepdims=True))

  Fin, Fout = Wf.shape
  return pl.pallas_call(
      body,
      grid=(_G,),
      in_specs=[
          pl.BlockSpec((_BLK, Fin), lambda i: (i, 0)),
          pl.BlockSpec((_BLK, Fin), lambda i: (i, 0)),
          pl.BlockSpec((_BLK, Fin), lambda i: (i, 0)),
          pl.BlockSpec((_BLK, 1), lambda i: (i, 0)),
          pl.BlockSpec((1, Fin), lambda i: (0, 0)),
          pl.BlockSpec((Fin, Fout), lambda i: (0, 0)),
          pl.BlockSpec((1, Fout), lambda i: (0, 0)),
      ],
      out_specs=pl.BlockSpec((_BLK, Fout), lambda i: (i, 0)),
      out_shape=jax.ShapeDtypeStruct((_N, Fout), jnp.float32),
  )(p0, p1, hlin, dinv, b, Wf, bf)


def kernel(x, edge_index, W1, b1, W2, b2, W3, b3, Wf, bf):
  src2 = edge_index[0].reshape(_E // _CH, _CH)
  dst2 = edge_index[1].reshape(_E // _CH, _CH)

  degp = _sc_degree(dst2)
  d0 = degp[0, :, :1]
  d1 = degp[1, :, :1]

  W3p = jnp.pad(W3, ((0, 0), (0, 4)))
  b3p = jnp.pad(b3, (0, 4))
  Wfp = jnp.pad(Wf, ((0, 4), (0, 0)))

  dinv, h1, g1 = _tc_first(x, W1, d0, d1)
  p = _sc_scatter(g1, src2, dst2, W1.shape[1])
  h2, g2 = _tc_mid(p[0], p[1], h1, dinv, b1.reshape(1, -1), W2)
  p = _sc_scatter(g2, src2, dst2, W2.shape[1])
  h3, g3 = _tc_mid(p[0], p[1], h2, dinv, b2.reshape(1, -1), W3p)
  p = _sc_scatter(g3, src2, dst2, 16)
  return _tc_final(p[0], p[1], h3, dinv, b3p.reshape(1, -1), Wfp,
                   bf.reshape(1, -1))


# speedup vs baseline: 51.3636x; 1.1640x over previous
import functools

import jax
import jax.numpy as jnp
from jax import lax
from jax.experimental import pallas as pl
from jax.experimental.pallas import tpu as pltpu
from jax.experimental.pallas import tpu_sc as plsc

_N = 50000
_D = 128
_E = 1600000

_NC = 2
_NS = 16
_NW = _NC * _NS
_CH = 125
_EPW = _E // _NW
_NCH = _EPW // _CH
_INNER = 8
_OUTER = _NCH // _INNER

_RPT = 3128
_RPT_LAST = _N - (_NS - 1) * _RPT

_BLK = 2000
_G = _N // _BLK


def _sc_mesh():
  return plsc.VectorSubcoreMesh(core_axis_name="c", subcore_axis_name="s")


def _sc_degree(dst2, W=8):
  @functools.partial(
      pl.kernel,
      out_type=jax.ShapeDtypeStruct((_NC, _N, W), jnp.float32),
      mesh=_sc_mesh(),
      scratch_types=[
          pltpu.VMEM((_INNER, _CH), jnp.int32),
          pltpu.VMEM((_CH, W), jnp.float32),
          pltpu.VMEM_SHARED((_N, W), jnp.float32),
          pltpu.SemaphoreType.DMA,
      ],
      compiler_params=pltpu.CompilerParams(use_tc_tiling_on_sc=False),
  )
  def k(dst_h, ones_h, zeros_h, out_h, didx, obuf, acc, ssem):
    c = lax.axis_index("c")
    s = lax.axis_index("s")
    wid = c * _NS + s

    @pl.when(s == 0)
    def _():
      pltpu.sync_copy(zeros_h, acc)

    pltpu.sync_copy(ones_h, obuf)
    plsc.subcore_barrier()

    def outer(o, carry):
      r0 = wid * _NCH + o * _INNER
      pltpu.sync_copy(dst_h.at[pl.ds(r0, _INNER)], didx)
      cps = [
          pltpu.async_copy(obuf, acc.at[didx.at[i]], ssem, add=True)
          for i in range(_INNER)
      ]
      for cp in cps:
        cp.wait()
      return carry

    lax.fori_loop(0, _OUTER, outer, 0)
    plsc.subcore_barrier()

    @pl.when(s == 0)
    def _():
      pltpu.sync_copy(acc, out_h.at[c])

  ones = jnp.ones((_CH, W), jnp.float32)
  zeros = jnp.zeros((_N, W), jnp.float32)
  return k(dst2, ones, zeros)


def _sc_scatter(g, src2, dst2, F):
  nbuf = 4 if F >= 32 else _INNER

  @functools.partial(
      pl.kernel,
      out_type=jax.ShapeDtypeStruct((_NC, _N, F), jnp.float32),
      mesh=_sc_mesh(),
      scratch_types=[
          pltpu.VMEM((_INNER, _CH), jnp.int32),
          pltpu.VMEM((_INNER, _CH), jnp.int32),
          pltpu.VMEM((nbuf, _CH, F), jnp.float32),
          pltpu.VMEM_SHARED((_N, F), jnp.float32),
          pltpu.SemaphoreType.DMA,
          pltpu.SemaphoreType.DMA,
      ],
      compiler_params=pltpu.CompilerParams(use_tc_tiling_on_sc=False),
  )
  def k(g_h, src_h, dst_h, zeros_h, out_h, sidx, didx, rows, acc, gsem, ssem):
    c = lax.axis_index("c")
    s = lax.axis_index("s")
    wid = c * _NS + s

    @pl.when(s < _NS - 1)
    def _():
      pltpu.sync_copy(zeros_h, acc.at[pl.ds(s * _RPT, _RPT)])

    @pl.when(s == _NS - 1)
    def _():
      pltpu.sync_copy(zeros_h.at[pl.ds(0, _RPT_LAST)],
                      acc.at[pl.ds((_NS - 1) * _RPT, _RPT_LAST)])

    plsc.subcore_barrier()

    def outer(o, carry):
      r0 = wid * _NCH + o * _INNER
      pltpu.sync_copy(src_h.at[pl.ds(r0, _INNER)], sidx)
      pltpu.sync_copy(dst_h.at[pl.ds(r0, _INNER)], didx)
      gcps = [None] * _INNER
      scps = [None] * _INNER
      for i in range(nbuf):
        gcps[i] = pltpu.async_copy(g_h.at[sidx.at[i]], rows.at[i % nbuf],
                                   gsem)
      for i in range(_INNER):
        gcps[i].wait()
        scps[i] = pltpu.async_copy(rows.at[i % nbuf], acc.at[didx.at[i]],
                                   ssem, add=True)
        nxt = i + nbuf
        if nxt < _INNER:
          scps[i].wait()
          gcps[nxt] = pltpu.async_copy(g_h.at[sidx.at[nxt]],
                                       rows.at[nxt % nbuf], gsem)
      for i in range(max(0, _INNER - nbuf), _INNER):
        scps[i].wait()
      return carry

    lax.fori_loop(0, _OUTER, outer, 0)
    plsc.subcore_barrier()

    @pl.when(s < _NS - 1)
    def _():
      pltpu.sync_copy(acc.at[pl.ds(s * _RPT, _RPT)],
                      out_h.at[c, pl.ds(s * _RPT, _RPT)])

    @pl.when(s == _NS - 1)
    def _():
      pltpu.sync_copy(acc.at[pl.ds((_NS - 1) * _RPT, _RPT_LAST)],
                      out_h.at[c, pl.ds((_NS - 1) * _RPT, _RPT_LAST)])

  zeros = jnp.zeros((_RPT, F), jnp.float32)
  return k(g, src2, dst2, zeros)


def _tc_matmul(x, W1):
  def body(x_r, w_r, h_r):
    h_r[...] = jnp.dot(x_r[...], w_r[...],
                       preferred_element_type=jnp.float32)

  F = W1.shape[1]
  return pl.pallas_call(
      body,
      grid=(_G,),
      in_specs=[
          pl.BlockSpec((_BLK, _D), lambda i: (i, 0)),
          pl.BlockSpec((_D, F), lambda i: (0, 0)),
      ],
      out_specs=pl.BlockSpec((_BLK, F), lambda i: (i, 0)),
      out_shape=jax.ShapeDtypeStruct((_N, F), jnp.float32),
  )(x, W1)


def _tc_prep(degp, h1):
  def body(dp_r, h_r, dinv_r, g_r):
    deg = 1.0 + dp_r[0, :, 0:1] + dp_r[1, :, 0:1]
    dinv = lax.rsqrt(deg)
    dinv_r[...] = dinv
    g_r[...] = h_r[...] * dinv

  F = h1.shape[1]
  W = degp.shape[2]
  return pl.pallas_call(
      body,
      grid=(_G,),
      in_specs=[
          pl.BlockSpec((2, _BLK, W), lambda i: (0, i, 0)),
          pl.BlockSpec((_BLK, F), lambda i: (i, 0)),
      ],
      out_specs=[
          pl.BlockSpec((_BLK, 1), lambda i: (i, 0)),
          pl.BlockSpec((_BLK, F), lambda i: (i, 0)),
      ],
      out_shape=[
          jax.ShapeDtypeStruct((_N, 1), jnp.float32),
          jax.ShapeDtypeStruct((_N, F), jnp.float32),
      ],
  )(degp, h1)


def _tc_mid(p, hlin, dinv, b, Wn):
  def body(p_r, h_r, di_r, b_r, w_r, h2_r, g2_r):
    dinv = di_r[...]
    h = jnp.maximum(
        dinv * (p_r[0] + p_r[1]) + (dinv * dinv) * h_r[...] + b_r[...],
        0.0)
    h2 = jnp.dot(h, w_r[...], preferred_element_type=jnp.float32)
    h2_r[...] = h2
    g2_r[...] = h2 * dinv

  Fin, Fout = Wn.shape
  return pl.pallas_call(
      body,
      grid=(_G,),
      in_specs=[
          pl.BlockSpec((2, _BLK, Fin), lambda i: (0, i, 0)),
          pl.BlockSpec((_BLK, Fin), lambda i: (i, 0)),
          pl.BlockSpec((_BLK, 1), lambda i: (i, 0)),
          pl.BlockSpec((1, Fin), lambda i: (0, 0)),
          pl.BlockSpec((Fin, Fout), lambda i: (0, 0)),
      ],
      out_specs=[
          pl.BlockSpec((_BLK, Fout), lambda i: (i, 0)),
          pl.BlockSpec((_BLK, Fout), lambda i: (i, 0)),
      ],
      out_shape=[
          jax.ShapeDtypeStruct((_N, Fout), jnp.float32),
          jax.ShapeDtypeStruct((_N, Fout), jnp.float32),
      ],
  )(p, hlin, dinv, b, Wn)


def _tc_final(p, hlin, dinv, b, Wf, bf):
  def body(p_r, h_r, di_r, b_r, wf_r, bf_r, o_r):
    dinv = di_r[...]
    h = jnp.maximum(
        dinv * (p_r[0] + p_r[1]) + (dinv * dinv) * h_r[...] + b_r[...],
        0.0)
    logits = jnp.dot(h, wf_r[...], preferred_element_type=jnp.float32)
    logits = logits + bf_r[...]
    m = jnp.max(logits, axis=1, keepdims=True)
    ex = jnp.exp(logits - m)
    o_r[...] = logits - m - jnp.log(jnp.sum(ex, axis=1, keepdims=True))

  Fin, Fout = Wf.shape
  return pl.pallas_call(
      body,
      grid=(_G,),
      in_specs=[
          pl.BlockSpec((2, _BLK, Fin), lambda i: (0, i, 0)),
          pl.BlockSpec((_BLK, Fin), lambda i: (i, 0)),
          pl.BlockSpec((_BLK, 1), lambda i: (i, 0)),
          pl.BlockSpec((1, Fin), lambda i: (0, 0)),
          pl.BlockSpec((Fin, Fout), lambda i: (0, 0)),
          pl.BlockSpec((1, Fout), lambda i: (0, 0)),
      ],
      out_specs=pl.BlockSpec((_BLK, Fout), lambda i: (i, 0)),
      out_shape=jax.ShapeDtypeStruct((_N, Fout), jnp.float32),
  )(p, hlin, dinv, b, Wf, bf)


def kernel(x, edge_index, W1, b1, W2, b2, W3, b3, Wf, bf):
  src2 = edge_index[0].reshape(_E // _CH, _CH)
  dst2 = edge_index[1].reshape(_E // _CH, _CH)

  W3p = jnp.pad(W3, ((0, 0), (0, 4)))
  b3p = jnp.pad(b3, (0, 4))
  Wfp = jnp.pad(Wf, ((0, 4), (0, 0)))

  degp = _sc_degree(dst2)
  h1 = _tc_matmul(x, W1)
  dinv, g1 = _tc_prep(degp, h1)
  p = _sc_scatter(g1, src2, dst2, W1.shape[1])
  h2, g2 = _tc_mid(p, h1, dinv, b1.reshape(1, -1), W2)
  p = _sc_scatter(g2, src2, dst2, W2.shape[1])
  h3, g3 = _tc_mid(p, h2, dinv, b2.reshape(1, -1), W3p)
  p = _sc_scatter(g3, src2, dst2, 16)
  return _tc_final(p, h3, dinv, b3p.reshape(1, -1), Wfp,
                   bf.reshape(1, -1))


# speedup vs baseline: 59.8613x; 1.1654x over previous
import functools

import jax
import jax.numpy as jnp
from jax import lax
from jax.experimental import pallas as pl
from jax.experimental.pallas import tpu as pltpu
from jax.experimental.pallas import tpu_sc as plsc

_N = 50000
_D = 128
_E = 1600000

_NC = 2
_NS = 16
_NW = _NC * _NS
_CH = 250
_EPW = _E // _NW
_INNER = 8

_RPT = 3128
_RPT_LAST = _N - (_NS - 1) * _RPT

_BLK = 2000
_G = _N // _BLK


def _sc_mesh():
  return plsc.VectorSubcoreMesh(core_axis_name="c", subcore_axis_name="s")


def _sc_degree(ei3, W=8, ch=_CH):
  nch = _EPW // ch
  outer_n = nch // _INNER

  @functools.partial(
      pl.kernel,
      out_type=jax.ShapeDtypeStruct((_NC, _N, W), jnp.float32),
      mesh=_sc_mesh(),
      scratch_types=[
          pltpu.VMEM((_INNER, ch), jnp.int32),
          pltpu.VMEM((ch, W), jnp.float32),
          pltpu.VMEM_SHARED((_N, W), jnp.float32),
          pltpu.SemaphoreType.DMA,
      ],
      compiler_params=pltpu.CompilerParams(use_tc_tiling_on_sc=False),
  )
  def k(ei_h, ones_h, zeros_h, out_h, didx, obuf, acc, ssem):
    c = lax.axis_index("c")
    s = lax.axis_index("s")
    wid = c * _NS + s

    @pl.when(s == 0)
    def _():
      pltpu.sync_copy(zeros_h, acc)

    pltpu.sync_copy(ones_h, obuf)
    plsc.subcore_barrier()

    def outer(o, carry):
      r0 = wid * nch + o * _INNER
      pltpu.sync_copy(ei_h.at[1, pl.ds(r0, _INNER)], didx)
      cps = [
          pltpu.async_copy(obuf, acc.at[didx.at[i]], ssem, add=True)
          for i in range(_INNER)
      ]
      for cp in cps:
        cp.wait()
      return carry

    lax.fori_loop(0, outer_n, outer, 0)
    plsc.subcore_barrier()

    @pl.when(s == 0)
    def _():
      pltpu.sync_copy(acc, out_h.at[c])

  ones = jnp.ones((ch, W), jnp.float32)
  zeros = jnp.zeros((_N, W), jnp.float32)
  return k(ei3, ones, zeros)


def _sc_scatter(g, ei3, F, ch=_CH, nbuf=None):
  if nbuf is None:
    nbuf = 3 if F >= 32 else _INNER
  nch = _EPW // ch
  outer_n = nch // _INNER

  @functools.partial(
      pl.kernel,
      out_type=jax.ShapeDtypeStruct((_NC, _N, F), jnp.float32),
      mesh=_sc_mesh(),
      scratch_types=[
          pltpu.VMEM((_INNER, ch), jnp.int32),
          pltpu.VMEM((_INNER, ch), jnp.int32),
          pltpu.VMEM((nbuf, ch, F), jnp.float32),
          pltpu.VMEM_SHARED((_N, F), jnp.float32),
          pltpu.SemaphoreType.DMA,
          pltpu.SemaphoreType.DMA,
      ],
      compiler_params=pltpu.CompilerParams(use_tc_tiling_on_sc=False),
  )
  def k(g_h, ei_h, zeros_h, out_h, sidx, didx, rows, acc, gsem, ssem):
    c = lax.axis_index("c")
    s = lax.axis_index("s")
    wid = c * _NS + s

    @pl.when(s < _NS - 1)
    def _():
      pltpu.sync_copy(zeros_h, acc.at[pl.ds(s * _RPT, _RPT)])

    @pl.when(s == _NS - 1)
    def _():
      pltpu.sync_copy(zeros_h.at[pl.ds(0, _RPT_LAST)],
                      acc.at[pl.ds((_NS - 1) * _RPT, _RPT_LAST)])

    plsc.subcore_barrier()

    def outer(o, carry):
      r0 = wid * nch + o * _INNER
      pltpu.sync_copy(ei_h.at[0, pl.ds(r0, _INNER)], sidx)
      pltpu.sync_copy(ei_h.at[1, pl.ds(r0, _INNER)], didx)
      gcps = [None] * _INNER
      scps = [None] * _INNER
      for i in range(nbuf):
        gcps[i] = pltpu.async_copy(g_h.at[sidx.at[i]], rows.at[i % nbuf],
                                   gsem)
      for i in range(_INNER):
        gcps[i].wait()
        scps[i] = pltpu.async_copy(rows.at[i % nbuf], acc.at[didx.at[i]],
                                   ssem, add=True)
        nxt = i + nbuf
        if nxt < _INNER:
          scps[i].wait()
          gcps[nxt] = pltpu.async_copy(g_h.at[sidx.at[nxt]],
                                       rows.at[nxt % nbuf], gsem)
      for i in range(max(0, _INNER - nbuf), _INNER):
        scps[i].wait()
      return carry

    lax.fori_loop(0, outer_n, outer, 0)
    plsc.subcore_barrier()

    @pl.when(s < _NS - 1)
    def _():
      pltpu.sync_copy(acc.at[pl.ds(s * _RPT, _RPT)],
                      out_h.at[c, pl.ds(s * _RPT, _RPT)])

    @pl.when(s == _NS - 1)
    def _():
      pltpu.sync_copy(acc.at[pl.ds((_NS - 1) * _RPT, _RPT_LAST)],
                      out_h.at[c, pl.ds((_NS - 1) * _RPT, _RPT_LAST)])

  zeros = jnp.zeros((_RPT, F), jnp.float32)
  return k(g, ei3, zeros)


def _tc_matmul(x, W1):
  def body(x_r, w_r, h_r):
    h_r[...] = jnp.dot(x_r[...], w_r[...],
                       preferred_element_type=jnp.float32)

  F = W1.shape[1]
  return pl.pallas_call(
      body,
      grid=(_G,),
      in_specs=[
          pl.BlockSpec((_BLK, _D), lambda i: (i, 0)),
          pl.BlockSpec((_D, F), lambda i: (0, 0)),
      ],
      out_specs=pl.BlockSpec((_BLK, F), lambda i: (i, 0)),
      out_shape=jax.ShapeDtypeStruct((_N, F), jnp.float32),
  )(x, W1)


def _tc_prep(degp, h1):
  def body(dp_r, h_r, dinv_r, g_r):
    deg = 1.0 + dp_r[0, :, 0:1] + dp_r[1, :, 0:1]
    dinv = lax.rsqrt(deg)
    dinv_r[...] = dinv
    g_r[...] = h_r[...] * dinv

  F = h1.shape[1]
  W = degp.shape[2]
  return pl.pallas_call(
      body,
      grid=(_G,),
      in_specs=[
          pl.BlockSpec((2, _BLK, W), lambda i: (0, i, 0)),
          pl.BlockSpec((_BLK, F), lambda i: (i, 0)),
      ],
      out_specs=[
          pl.BlockSpec((_BLK, 1), lambda i: (i, 0)),
          pl.BlockSpec((_BLK, F), lambda i: (i, 0)),
      ],
      out_shape=[
          jax.ShapeDtypeStruct((_N, 1), jnp.float32),
          jax.ShapeDtypeStruct((_N, F), jnp.float32),
      ],
  )(degp, h1)


def _tc_mid(p, hlin, dinv, b, Wn):
  def body(p_r, h_r, di_r, b_r, w_r, h2_r, g2_r):
    dinv = di_r[...]
    h = jnp.maximum(
        dinv * (p_r[0] + p_r[1]) + (dinv * dinv) * h_r[...] + b_r[...],
        0.0)
    h2 = jnp.dot(h, w_r[...], preferred_element_type=jnp.float32)
    h2_r[...] = h2
    g2_r[...] = h2 * dinv

  Fin, Fout = Wn.shape
  return pl.pallas_call(
      body,
      grid=(_G,),
      in_specs=[
          pl.BlockSpec((2, _BLK, Fin), lambda i: (0, i, 0)),
          pl.BlockSpec((_BLK, Fin), lambda i: (i, 0)),
          pl.BlockSpec((_BLK, 1), lambda i: (i, 0)),
          pl.BlockSpec((1, Fin), lambda i: (0, 0)),
          pl.BlockSpec((Fin, Fout), lambda i: (0, 0)),
      ],
      out_specs=[
          pl.BlockSpec((_BLK, Fout), lambda i: (i, 0)),
          pl.BlockSpec((_BLK, Fout), lambda i: (i, 0)),
      ],
      out_shape=[
          jax.ShapeDtypeStruct((_N, Fout), jnp.float32),
          jax.ShapeDtypeStruct((_N, Fout), jnp.float32),
      ],
  )(p, hlin, dinv, b, Wn)


def _tc_final(p, hlin, dinv, b, Wf, bf):
  def body(p_r, h_r, di_r, b_r, wf_r, bf_r, o_r):
    dinv = di_r[...]
    h = jnp.maximum(
        dinv * (p_r[0] + p_r[1]) + (dinv * dinv) * h_r[...] + b_r[...],
        0.0)
    logits = jnp.dot(h, wf_r[...], preferred_element_type=jnp.float32)
    logits = logits + bf_r[...]
    m = jnp.max(logits, axis=1, keepdims=True)
    ex = jnp.exp(logits - m)
    o_r[...] = logits - m - jnp.log(jnp.sum(ex, axis=1, keepdims=True))

  Fin, Fout = Wf.shape
  return pl.pallas_call(
      body,
      grid=(_G,),
      in_specs=[
          pl.BlockSpec((2, _BLK, Fin), lambda i: (0, i, 0)),
          pl.BlockSpec((_BLK, Fin), lambda i: (i, 0)),
          pl.BlockSpec((_BLK, 1), lambda i: (i, 0)),
          pl.BlockSpec((1, Fin), lambda i: (0, 0)),
          pl.BlockSpec((Fin, Fout), lambda i: (0, 0)),
          pl.BlockSpec((1, Fout), lambda i: (0, 0)),
      ],
      out_specs=pl.BlockSpec((_BLK, Fout), lambda i: (i, 0)),
      out_shape=jax.ShapeDtypeStruct((_N, Fout), jnp.float32),
  )(p, hlin, dinv, b, Wf, bf)


def kernel(x, edge_index, W1, b1, W2, b2, W3, b3, Wf, bf):
  ei3 = edge_index.reshape(2, _E // _CH, _CH)

  W3p = jnp.pad(W3, ((0, 0), (0, 4)))
  b3p = jnp.pad(b3, (0, 4))
  Wfp = jnp.pad(Wf, ((0, 4), (0, 0)))

  degp = _sc_degree(ei3)
  h1 = _tc_matmul(x, W1)
  dinv, g1 = _tc_prep(degp, h1)
  p = _sc_scatter(g1, ei3, W1.shape[1])
  h2, g2 = _tc_mid(p, h1, dinv, b1.reshape(1, -1), W2)
  p = _sc_scatter(g2, ei3, W2.shape[1])
  h3, g3 = _tc_mid(p, h2, dinv, b2.reshape(1, -1), W3p)
  p = _sc_scatter(g3, ei3, 16)
  return _tc_final(p, h3, dinv, b3p.reshape(1, -1), Wfp,
                   bf.reshape(1, -1))


# speedup vs baseline: 60.2887x; 1.0071x over previous
import functools

import jax
import jax.numpy as jnp
from jax import lax
from jax.experimental import pallas as pl
from jax.experimental.pallas import tpu as pltpu
from jax.experimental.pallas import tpu_sc as plsc

_N = 50000
_D = 128
_E = 1600000

_NC = 2
_NS = 16
_NW = _NC * _NS
_CH = 125
_EPW = _E // _NW
_INNER = 8

_RPT = 3128
_RPT_LAST = _N - (_NS - 1) * _RPT

_BLK = 2000
_G = _N // _BLK


def _sc_mesh():
  return plsc.VectorSubcoreMesh(core_axis_name="c", subcore_axis_name="s")


def _sc_degree(ei3, W=8, ch=_CH):
  nch = _EPW // ch
  outer_n = nch // _INNER

  @functools.partial(
      pl.kernel,
      out_type=jax.ShapeDtypeStruct((_NC, _N, W), jnp.float32),
      mesh=_sc_mesh(),
      scratch_types=[
          pltpu.VMEM((_INNER, ch), jnp.int32),
          pltpu.VMEM((ch, W), jnp.float32),
          pltpu.VMEM_SHARED((_N, W), jnp.float32),
          pltpu.SemaphoreType.DMA,
      ],
      compiler_params=pltpu.CompilerParams(use_tc_tiling_on_sc=False),
  )
  def k(ei_h, ones_h, zeros_h, out_h, didx, obuf, acc, ssem):
    c = lax.axis_index("c")
    s = lax.axis_index("s")
    wid = c * _NS + s

    @pl.when(s == 0)
    def _():
      pltpu.sync_copy(zeros_h, acc)

    pltpu.sync_copy(ones_h, obuf)
    plsc.subcore_barrier()

    def outer(o, carry):
      r0 = wid * nch + o * _INNER
      pltpu.sync_copy(ei_h.at[1, pl.ds(r0, _INNER)], didx)
      cps = [
          pltpu.async_copy(obuf, acc.at[didx.at[i]], ssem, add=True)
          for i in range(_INNER)
      ]
      for cp in cps:
        cp.wait()
      return carry

    lax.fori_loop(0, outer_n, outer, 0)
    plsc.subcore_barrier()

    @pl.when(s == 0)
    def _():
      pltpu.sync_copy(acc, out_h.at[c])

  ones = jnp.ones((ch, W), jnp.float32)
  zeros = jnp.zeros((_N, W), jnp.float32)
  return k(ei3, ones, zeros)


def _sc_scatter(g, ei3, F, ch=None, nbuf=4):
  if ch is None:
    ch = _CH
  nch = _EPW // ch
  outer_n = nch // _INNER
  assert _INNER % nbuf == 0 and nch % _INNER == 0

  @functools.partial(
      pl.kernel,
      out_type=jax.ShapeDtypeStruct((_NC, _N, F), jnp.float32),
      mesh=_sc_mesh(),
      scratch_types=[
          pltpu.VMEM((2, _INNER, ch), jnp.int32),
          pltpu.VMEM((2, _INNER, ch), jnp.int32),
          pltpu.VMEM((nbuf, ch, F), jnp.float32),
          pltpu.VMEM_SHARED((_N, F), jnp.float32),
          pltpu.SemaphoreType.DMA,
          pltpu.SemaphoreType.DMA,
          pltpu.SemaphoreType.DMA,
      ],
      compiler_params=pltpu.CompilerParams(use_tc_tiling_on_sc=False),
  )
  def k(g_h, ei_h, zeros_h, out_h, sidx, didx, rows, acc, gsem, ssem, isem):
    c = lax.axis_index("c")
    s = lax.axis_index("s")
    wid = c * _NS + s
    base = wid * nch

    @pl.when(s < _NS - 1)
    def _():
      pltpu.sync_copy(zeros_h, acc.at[pl.ds(s * _RPT, _RPT)])

    @pl.when(s == _NS - 1)
    def _():
      pltpu.sync_copy(zeros_h.at[pl.ds(0, _RPT_LAST)],
                      acc.at[pl.ds((_NS - 1) * _RPT, _RPT_LAST)])

    plsc.subcore_barrier()

    drow = zeros_h.at[pl.ds(0, ch)]
    dsrc = ei_h.at[0, pl.ds(0, _INNER)]

    pltpu.sync_copy(ei_h.at[0, pl.ds(base, _INNER)], sidx.at[0])
    pltpu.sync_copy(ei_h.at[1, pl.ds(base, _INNER)], didx.at[0])
    for i in range(nbuf):
      pltpu.async_copy(g_h.at[sidx.at[0, i]], rows.at[i], gsem)

    def outer(o, carry):
      slot = lax.rem(o, 2)
      nslot = lax.rem(o + 1, 2)
      more = o + 1 < outer_n

      @pl.when(more)
      def _():
        nb = base + (o + 1) * _INNER
        pltpu.async_copy(ei_h.at[0, pl.ds(nb, _INNER)], sidx.at[nslot], isem)
        pltpu.async_copy(ei_h.at[1, pl.ds(nb, _INNER)], didx.at[nslot], isem)

      for i in range(_INNER):
        b = i % nbuf
        pltpu.make_async_copy(drow, rows.at[b], gsem).wait()
        pltpu.async_copy(rows.at[b], acc.at[didx.at[slot, i]], ssem,
                         add=True)
        if i == _INNER - nbuf:
          @pl.when(more)
          def _():
            pltpu.make_async_copy(dsrc, sidx.at[0], isem).wait()
            pltpu.make_async_copy(dsrc, didx.at[0], isem).wait()
        ahead = i + nbuf
        if ahead < _INNER:
          @pl.when(o * _INNER + ahead < nch)
          def _():
            pltpu.make_async_copy(drow, rows.at[b], ssem).wait()
            pltpu.async_copy(g_h.at[sidx.at[slot, ahead]], rows.at[b], gsem)
        else:
          @pl.when(more)
          def _():
            pltpu.make_async_copy(drow, rows.at[b], ssem).wait()
            pltpu.async_copy(g_h.at[sidx.at[nslot, ahead - _INNER]],
                             rows.at[b], gsem)
      return carry

    lax.fori_loop(0, outer_n, outer, 0)
    for _ in range(nbuf):
      pltpu.make_async_copy(drow, rows.at[0], ssem).wait()
    plsc.subcore_barrier()

    @pl.when(s < _NS - 1)
    def _():
      pltpu.sync_copy(acc.at[pl.ds(s * _RPT, _RPT)],
                      out_h.at[c, pl.ds(s * _RPT, _RPT)])

    @pl.when(s == _NS - 1)
    def _():
      pltpu.sync_copy(acc.at[pl.ds((_NS - 1) * _RPT, _RPT_LAST)],
                      out_h.at[c, pl.ds((_NS - 1) * _RPT, _RPT_LAST)])

  zeros = jnp.zeros((_RPT, F), jnp.float32)
  return k(g, ei3, zeros)


def _tc_matmul(x, W1):
  def body(x_r, w_r, h_r):
    h_r[...] = jnp.dot(x_r[...], w_r[...],
                       preferred_element_type=jnp.float32)

  F = W1.shape[1]
  return pl.pallas_call(
      body,
      grid=(_G,),
      in_specs=[
          pl.BlockSpec((_BLK, _D), lambda i: (i, 0)),
          pl.BlockSpec((_D, F), lambda i: (0, 0)),
      ],
      out_specs=pl.BlockSpec((_BLK, F), lambda i: (i, 0)),
      out_shape=jax.ShapeDtypeStruct((_N, F), jnp.float32),
  )(x, W1)


def _tc_prep(degp, h1):
  def body(dp_r, h_r, dinv_r, g_r):
    deg = 1.0 + dp_r[0, :, 0:1] + dp_r[1, :, 0:1]
    dinv = lax.rsqrt(deg)
    dinv_r[...] = dinv
    g_r[...] = h_r[...] * dinv

  F = h1.shape[1]
  W = degp.shape[2]
  return pl.pallas_call(
      body,
      grid=(_G,),
      in_specs=[
          pl.BlockSpec((2, _BLK, W), lambda i: (0, i, 0)),
          pl.BlockSpec((_BLK, F), lambda i: (i, 0)),
      ],
      out_specs=[
          pl.BlockSpec((_BLK, 1), lambda i: (i, 0)),
          pl.BlockSpec((_BLK, F), lambda i: (i, 0)),
      ],
      out_shape=[
          jax.ShapeDtypeStruct((_N, 1), jnp.float32),
          jax.ShapeDtypeStruct((_N, F), jnp.float32),
      ],
  )(degp, h1)


def _tc_mid(p, hlin, dinv, b, Wn):
  def body(p_r, h_r, di_r, b_r, w_r, h2_r, g2_r):
    dinv = di_r[...]
    h = jnp.maximum(
        dinv * (p_r[0] + p_r[1]) + (dinv * dinv) * h_r[...] + b_r[...],
        0.0)
    h2 = jnp.dot(h, w_r[...], preferred_element_type=jnp.float32)
    h2_r[...] = h2
    g2_r[...] = h2 * dinv

  Fin, Fout = Wn.shape
  return pl.pallas_call(
      body,
      grid=(_G,),
      in_specs=[
          pl.BlockSpec((2, _BLK, Fin), lambda i: (0, i, 0)),
          pl.BlockSpec((_BLK, Fin), lambda i: (i, 0)),
          pl.BlockSpec((_BLK, 1), lambda i: (i, 0)),
          pl.BlockSpec((1, Fin), lambda i: (0, 0)),
          pl.BlockSpec((Fin, Fout), lambda i: (0, 0)),
      ],
      out_specs=[
          pl.BlockSpec((_BLK, Fout), lambda i: (i, 0)),
          pl.BlockSpec((_BLK, Fout), lambda i: (i, 0)),
      ],
      out_shape=[
          jax.ShapeDtypeStruct((_N, Fout), jnp.float32),
          jax.ShapeDtypeStruct((_N, Fout), jnp.float32),
      ],
  )(p, hlin, dinv, b, Wn)


def _tc_final(p, hlin, dinv, b, Wf, bf):
  def body(p_r, h_r, di_r, b_r, wf_r, bf_r, o_r):
    dinv = di_r[...]
    h = jnp.maximum(
        dinv * (p_r[0] + p_r[1]) + (dinv * dinv) * h_r[...] + b_r[...],
        0.0)
    logits = jnp.dot(h, wf_r[...], preferred_element_type=jnp.float32)
    logits = logits + bf_r[...]
    m = jnp.max(logits, axis=1, keepdims=True)
    ex = jnp.exp(logits - m)
    o_r[...] = logits - m - jnp.log(jnp.sum(ex, axis=1, keepdims=True))

  Fin, Fout = Wf.shape
  return pl.pallas_call(
      body,
      grid=(_G,),
      in_specs=[
          pl.BlockSpec((2, _BLK, Fin), lambda i: (0, i, 0)),
          pl.BlockSpec((_BLK, Fin), lambda i: (i, 0)),
          pl.BlockSpec((_BLK, 1), lambda i: (i, 0)),
          pl.BlockSpec((1, Fin), lambda i: (0, 0)),
          pl.BlockSpec((Fin, Fout), lambda i: (0, 0)),
          pl.BlockSpec((1, Fout), lambda i: (0, 0)),
      ],
      out_specs=pl.BlockSpec((_BLK, Fout), lambda i: (i, 0)),
      out_shape=jax.ShapeDtypeStruct((_N, Fout), jnp.float32),
  )(p, hlin, dinv, b, Wf, bf)


def kernel(x, edge_index, W1, b1, W2, b2, W3, b3, Wf, bf):
  ei3 = edge_index.reshape(2, _E // _CH, _CH)

  W3p = jnp.pad(W3, ((0, 0), (0, 4)))
  b3p = jnp.pad(b3, (0, 4))
  Wfp = jnp.pad(Wf, ((0, 4), (0, 0)))

  degp = _sc_degree(ei3)
  h1 = _tc_matmul(x, W1)
  dinv, g1 = _tc_prep(degp, h1)
  p = _sc_scatter(g1, ei3, W1.shape[1])
  h2, g2 = _tc_mid(p, h1, dinv, b1.reshape(1, -1), W2)
  p = _sc_scatter(g2, ei3, W2.shape[1])
  h3, g3 = _tc_mid(p, h2, dinv, b2.reshape(1, -1), W3p)
  p = _sc_scatter(g3, ei3, 16)
  return _tc_final(p, h3, dinv, b3p.reshape(1, -1), Wfp,
                   bf.reshape(1, -1))


# speedup vs baseline: 67.9690x; 1.1274x over previous
import functools

import jax
import jax.numpy as jnp
from jax import lax
from jax.experimental import pallas as pl
from jax.experimental.pallas import tpu as pltpu
from jax.experimental.pallas import tpu_sc as plsc

_N = 50000
_D = 128
_E = 1600000

_NC = 2
_NS = 16
_NW = _NC * _NS
_CH = 125
_EPW = _E // _NW
_INNER = 8

_RPT = 3128
_RPT_LAST = _N - (_NS - 1) * _RPT

_BLK = 2000
_G = _N // _BLK


def _sc_mesh():
  return plsc.VectorSubcoreMesh(core_axis_name="c", subcore_axis_name="s")


def _sc_degree(ei, W=8, ch=2000, inner=5):
  nch = _EPW // ch
  outer_n = nch // inner
  blk = inner * ch

  @functools.partial(
      pl.kernel,
      out_type=jax.ShapeDtypeStruct((_NC, _N, W), jnp.float32),
      mesh=_sc_mesh(),
      scratch_types=[
          pltpu.VMEM((2, blk), jnp.int32),
          pltpu.VMEM((ch, W), jnp.float32),
          pltpu.VMEM_SHARED((_N, W), jnp.float32),
          pltpu.SemaphoreType.DMA,
          pltpu.SemaphoreType.DMA,
      ],
      compiler_params=pltpu.CompilerParams(use_tc_tiling_on_sc=False),
  )
  def k(ei_h, ones_h, zeros_h, out_h, didx, obuf, acc, ssem, isem):
    c = lax.axis_index("c")
    s = lax.axis_index("s")
    wid = c * _NS + s
    base = wid * _EPW

    @pl.when(s < _NS - 1)
    def _():
      pltpu.sync_copy(zeros_h, acc.at[pl.ds(s * _RPT, _RPT)])

    @pl.when(s == _NS - 1)
    def _():
      pltpu.sync_copy(zeros_h.at[pl.ds(0, _RPT_LAST)],
                      acc.at[pl.ds((_NS - 1) * _RPT, _RPT_LAST)])

    pltpu.sync_copy(ones_h, obuf)
    plsc.subcore_barrier()

    dobuf = zeros_h.at[pl.ds(0, ch)]
    dsrc = ei_h.at[1, pl.ds(0, blk)]

    pltpu.sync_copy(ei_h.at[1, pl.ds(base, blk)], didx.at[0])

    def outer(o, carry):
      slot = lax.rem(o, 2)
      nslot = lax.rem(o + 1, 2)

      @pl.when(o >= 1)
      def _():
        for _ in range(inner):
          pltpu.make_async_copy(dobuf, obuf, ssem).wait()
        pltpu.make_async_copy(dsrc, didx.at[0], isem).wait()

      @pl.when(o + 1 < outer_n)
      def _():
        nb = base + (o + 1) * blk
        pltpu.async_copy(ei_h.at[1, pl.ds(nb, blk)], didx.at[nslot], isem)

      for i in range(inner):
        pltpu.async_copy(obuf,
                         acc.at[didx.at[slot, pl.ds(i * ch, ch)]], ssem,
                         add=True)
      return carry

    lax.fori_loop(0, outer_n, outer, 0)
    for _ in range(inner):
      pltpu.make_async_copy(dobuf, obuf, ssem).wait()
    plsc.subcore_barrier()

    @pl.when(s < _NS - 1)
    def _():
      pltpu.sync_copy(acc.at[pl.ds(s * _RPT, _RPT)],
                      out_h.at[c, pl.ds(s * _RPT, _RPT)])

    @pl.when(s == _NS - 1)
    def _():
      pltpu.sync_copy(acc.at[pl.ds((_NS - 1) * _RPT, _RPT_LAST)],
                      out_h.at[c, pl.ds((_NS - 1) * _RPT, _RPT_LAST)])

  ones = jnp.ones((ch, W), jnp.float32)
  zeros = jnp.zeros((_RPT, W), jnp.float32)
  return k(ei, ones, zeros)


def _sc_scatter(g, ei, F, ch=200, nbuf=None, inner=10):
  if nbuf is None:
    nbuf = 2 if F >= 32 else 5
  nch = _EPW // ch
  outer_n = nch // inner
  blk = inner * ch
  assert inner % nbuf == 0 and nch % inner == 0 and ch % 8 == 0

  @functools.partial(
      pl.kernel,
      out_type=jax.ShapeDtypeStruct((_NC, _N, F), jnp.float32),
      mesh=_sc_mesh(),
      scratch_types=[
          pltpu.VMEM((2, blk), jnp.int32),
          pltpu.VMEM((2, blk), jnp.int32),
          pltpu.VMEM((nbuf, ch, F), jnp.float32),
          pltpu.VMEM_SHARED((_N, F), jnp.float32),
          pltpu.SemaphoreType.DMA,
          pltpu.SemaphoreType.DMA,
          pltpu.SemaphoreType.DMA,
      ],
      compiler_params=pltpu.CompilerParams(use_tc_tiling_on_sc=False),
  )
  def k(g_h, ei_h, zeros_h, out_h, sidx, didx, rows, acc, gsem, ssem, isem):
    c = lax.axis_index("c")
    s = lax.axis_index("s")
    wid = c * _NS + s
    base = wid * _EPW

    @pl.when(s < _NS - 1)
    def _():
      pltpu.sync_copy(zeros_h, acc.at[pl.ds(s * _RPT, _RPT)])

    @pl.when(s == _NS - 1)
    def _():
      pltpu.sync_copy(zeros_h.at[pl.ds(0, _RPT_LAST)],
                      acc.at[pl.ds((_NS - 1) * _RPT, _RPT_LAST)])

    plsc.subcore_barrier()

    drow = zeros_h.at[pl.ds(0, ch)]
    dsrc = ei_h.at[0, pl.ds(0, blk)]

    pltpu.sync_copy(ei_h.at[0, pl.ds(base, blk)], sidx.at[0])
    pltpu.sync_copy(ei_h.at[1, pl.ds(base, blk)], didx.at[0])
    for i in range(nbuf):
      pltpu.async_copy(g_h.at[sidx.at[0, pl.ds(i * ch, ch)]], rows.at[i],
                       gsem)

    def outer(o, carry):
      slot = lax.rem(o, 2)
      nslot = lax.rem(o + 1, 2)
      more = o + 1 < outer_n

      @pl.when(more)
      def _():
        nb = base + (o + 1) * blk
        pltpu.async_copy(ei_h.at[0, pl.ds(nb, blk)], sidx.at[nslot], isem)
        pltpu.async_copy(ei_h.at[1, pl.ds(nb, blk)], didx.at[nslot], isem)

      for i in range(inner):
        b = i % nbuf
        pltpu.make_async_copy(drow, rows.at[b], gsem).wait()
        pltpu.async_copy(rows.at[b],
                         acc.at[didx.at[slot, pl.ds(i * ch, ch)]], ssem,
                         add=True)
        if i == inner - nbuf:
          @pl.when(more)
          def _():
            pltpu.make_async_copy(dsrc, sidx.at[0], isem).wait()
            pltpu.make_async_copy(dsrc, didx.at[0], isem).wait()
        ahead = i + nbuf
        if ahead < inner:
          pltpu.make_async_copy(drow, rows.at[b], ssem).wait()
          pltpu.async_copy(g_h.at[sidx.at[slot, pl.ds(ahead * ch, ch)]],
                           rows.at[b], gsem)
        else:
          @pl.when(more)
          def _():
            pltpu.make_async_copy(drow, rows.at[b], ssem).wait()
            pltpu.async_copy(
                g_h.at[sidx.at[nslot, pl.ds((ahead - inner) * ch, ch)]],
                rows.at[b], gsem)
      return carry

    lax.fori_loop(0, outer_n, outer, 0)
    for _ in range(nbuf):
      pltpu.make_async_copy(drow, rows.at[0], ssem).wait()
    plsc.subcore_barrier()

    @pl.when(s < _NS - 1)
    def _():
      pltpu.sync_copy(acc.at[pl.ds(s * _RPT, _RPT)],
                      out_h.at[c, pl.ds(s * _RPT, _RPT)])

    @pl.when(s == _NS - 1)
    def _():
      pltpu.sync_copy(acc.at[pl.ds((_NS - 1) * _RPT, _RPT_LAST)],
                      out_h.at[c, pl.ds((_NS - 1) * _RPT, _RPT_LAST)])

  zeros = jnp.zeros((_RPT, F), jnp.float32)
  return k(g, ei, zeros)


def _tc_matmul(x, W1):
  def body(x_r, w_r, h_r):
    h_r[...] = jnp.dot(x_r[...], w_r[...],
                       preferred_element_type=jnp.float32)

  F = W1.shape[1]
  return pl.pallas_call(
      body,
      grid=(_G,),
      in_specs=[
          pl.BlockSpec((_BLK, _D), lambda i: (i, 0)),
          pl.BlockSpec((_D, F), lambda i: (0, 0)),
      ],
      out_specs=pl.BlockSpec((_BLK, F), lambda i: (i, 0)),
      out_shape=jax.ShapeDtypeStruct((_N, F), jnp.float32),
  )(x, W1)


def _tc_prep(degp, h1):
  def body(dp_r, h_r, dinv_r, g_r):
    deg = 1.0 + dp_r[0, :, 0:1] + dp_r[1, :, 0:1]
    dinv = lax.rsqrt(deg)
    dinv_r[...] = dinv
    g_r[...] = h_r[...] * dinv

  F = h1.shape[1]
  W = degp.shape[2]
  return pl.pallas_call(
      body,
      grid=(_G,),
      in_specs=[
          pl.BlockSpec((2, _BLK, W), lambda i: (0, i, 0)),
          pl.BlockSpec((_BLK, F), lambda i: (i, 0)),
      ],
      out_specs=[
          pl.BlockSpec((_BLK, 1), lambda i: (i, 0)),
          pl.BlockSpec((_BLK, F), lambda i: (i, 0)),
      ],
      out_shape=[
          jax.ShapeDtypeStruct((_N, 1), jnp.float32),
          jax.ShapeDtypeStruct((_N, F), jnp.float32),
      ],
  )(degp, h1)


def _tc_mid(p, hlin, dinv, b, Wn):
  def body(p_r, h_r, di_r, b_r, w_r, h2_r, g2_r):
    dinv = di_r[...]
    h = jnp.maximum(
        dinv * (p_r[0] + p_r[1]) + (dinv * dinv) * h_r[...] + b_r[...],
        0.0)
    h2 = jnp.dot(h, w_r[...], preferred_element_type=jnp.float32)
    h2_r[...] = h2
    g2_r[...] = h2 * dinv

  Fin, Fout = Wn.shape
  return pl.pallas_call(
      body,
      grid=(_G,),
      in_specs=[
          pl.BlockSpec((2, _BLK, Fin), lambda i: (0, i, 0)),
          pl.BlockSpec((_BLK, Fin), lambda i: (i, 0)),
          pl.BlockSpec((_BLK, 1), lambda i: (i, 0)),
          pl.BlockSpec((1, Fin), lambda i: (0, 0)),
          pl.BlockSpec((Fin, Fout), lambda i: (0, 0)),
      ],
      out_specs=[
          pl.BlockSpec((_BLK, Fout), lambda i: (i, 0)),
          pl.BlockSpec((_BLK, Fout), lambda i: (i, 0)),
      ],
      out_shape=[
          jax.ShapeDtypeStruct((_N, Fout), jnp.float32),
          jax.ShapeDtypeStruct((_N, Fout), jnp.float32),
      ],
  )(p, hlin, dinv, b, Wn)


def _tc_final(p, hlin, dinv, b, Wf, bf):
  def body(p_r, h_r, di_r, b_r, wf_r, bf_r, o_r):
    dinv = di_r[...]
    h = jnp.maximum(
        dinv * (p_r[0] + p_r[1]) + (dinv * dinv) * h_r[...] + b_r[...],
        0.0)
    logits = jnp.dot(h, wf_r[...], preferred_element_type=jnp.float32)
    logits = logits + bf_r[...]
    m = jnp.max(logits, axis=1, keepdims=True)
    ex = jnp.exp(logits - m)
    o_r[...] = logits - m - jnp.log(jnp.sum(ex, axis=1, keepdims=True))

  Fin, Fout = Wf.shape
  return pl.pallas_call(
      body,
      grid=(_G,),
      in_specs=[
          pl.BlockSpec((2, _BLK, Fin), lambda i: (0, i, 0)),
          pl.BlockSpec((_BLK, Fin), lambda i: (i, 0)),
          pl.BlockSpec((_BLK, 1), lambda i: (i, 0)),
          pl.BlockSpec((1, Fin), lambda i: (0, 0)),
          pl.BlockSpec((Fin, Fout), lambda i: (0, 0)),
          pl.BlockSpec((1, Fout), lambda i: (0, 0)),
      ],
      out_specs=pl.BlockSpec((_BLK, Fout), lambda i: (i, 0)),
      out_shape=jax.ShapeDtypeStruct((_N, Fout), jnp.float32),
  )(p, hlin, dinv, b, Wf, bf)


def kernel(x, edge_index, W1, b1, W2, b2, W3, b3, Wf, bf):
  W3p = jnp.pad(W3, ((0, 0), (0, 4)))
  b3p = jnp.pad(b3, (0, 4))
  Wfp = jnp.pad(Wf, ((0, 4), (0, 0)))

  degp = _sc_degree(edge_index)
  h1 = _tc_matmul(x, W1)
  dinv, g1 = _tc_prep(degp, h1)
  p = _sc_scatter(g1, edge_index, W1.shape[1])
  h2, g2 = _tc_mid(p, h1, dinv, b1.reshape(1, -1), W2)
  p = _sc_scatter(g2, edge_index, W2.shape[1])
  h3, g3 = _tc_mid(p, h2, dinv, b2.reshape(1, -1), W3p)
  p = _sc_scatter(g3, edge_index, 16)
  return _tc_final(p, h3, dinv, b3p.reshape(1, -1), Wfp,
                   bf.reshape(1, -1))


# speedup vs baseline: 70.6869x; 1.0400x over previous
import functools

import jax
import jax.numpy as jnp
from jax import lax
from jax.experimental import pallas as pl
from jax.experimental.pallas import tpu as pltpu
from jax.experimental.pallas import tpu_sc as plsc

_N = 50000
_D = 128
_E = 1600000

_NC = 2
_NS = 16
_NW = _NC * _NS
_CH = 125
_EPW = _E // _NW
_INNER = 8

_RPT = 3128
_RPT_LAST = _N - (_NS - 1) * _RPT

_BLK = 2000
_G = _N // _BLK


def _sc_mesh():
  return plsc.VectorSubcoreMesh(core_axis_name="c", subcore_axis_name="s")


def _sc_degree(ei, W=8, ch=2000, inner=5):
  nch = _EPW // ch
  outer_n = nch // inner
  blk = inner * ch

  @functools.partial(
      pl.kernel,
      out_type=jax.ShapeDtypeStruct((_NC, _N, W), jnp.float32),
      mesh=_sc_mesh(),
      scratch_types=[
          pltpu.VMEM((2, blk), jnp.int32),
          pltpu.VMEM((ch, W), jnp.float32),
          pltpu.VMEM_SHARED((_N, W), jnp.float32),
          pltpu.SemaphoreType.DMA,
          pltpu.SemaphoreType.DMA,
      ],
      compiler_params=pltpu.CompilerParams(use_tc_tiling_on_sc=False),
  )
  def k(ei_h, ones_h, zeros_h, out_h, didx, obuf, acc, ssem, isem):
    c = lax.axis_index("c")
    s = lax.axis_index("s")
    wid = c * _NS + s
    base = wid * _EPW

    @pl.when(s < _NS - 1)
    def _():
      pltpu.sync_copy(zeros_h, acc.at[pl.ds(s * _RPT, _RPT)])

    @pl.when(s == _NS - 1)
    def _():
      pltpu.sync_copy(zeros_h.at[pl.ds(0, _RPT_LAST)],
                      acc.at[pl.ds((_NS - 1) * _RPT, _RPT_LAST)])

    pltpu.sync_copy(ones_h, obuf)
    plsc.subcore_barrier()

    dobuf = zeros_h.at[pl.ds(0, ch)]
    dsrc = ei_h.at[1, pl.ds(0, blk)]

    pltpu.sync_copy(ei_h.at[1, pl.ds(base, blk)], didx.at[0])

    def outer(o, carry):
      slot = lax.rem(o, 2)
      nslot = lax.rem(o + 1, 2)

      @pl.when(o >= 1)
      def _():
        for _ in range(inner):
          pltpu.make_async_copy(dobuf, obuf, ssem).wait()
        pltpu.make_async_copy(dsrc, didx.at[0], isem).wait()

      @pl.when(o + 1 < outer_n)
      def _():
        nb = base + (o + 1) * blk
        pltpu.async_copy(ei_h.at[1, pl.ds(nb, blk)], didx.at[nslot], isem)

      for i in range(inner):
        pltpu.async_copy(obuf,
                         acc.at[didx.at[slot, pl.ds(i * ch, ch)]], ssem,
                         add=True)
      return carry

    lax.fori_loop(0, outer_n, outer, 0)
    for _ in range(inner):
      pltpu.make_async_copy(dobuf, obuf, ssem).wait()
    plsc.subcore_barrier()

    @pl.when(s < _NS - 1)
    def _():
      pltpu.sync_copy(acc.at[pl.ds(s * _RPT, _RPT)],
                      out_h.at[c, pl.ds(s * _RPT, _RPT)])

    @pl.when(s == _NS - 1)
    def _():
      pltpu.sync_copy(acc.at[pl.ds((_NS - 1) * _RPT, _RPT_LAST)],
                      out_h.at[c, pl.ds((_NS - 1) * _RPT, _RPT_LAST)])

  ones = jnp.ones((ch, W), jnp.float32)
  zeros = jnp.zeros((_RPT, W), jnp.float32)
  return k(ei, ones, zeros)


def _sc_scatter(g, ei, F, ch=None, nbuf=None, inner=None):
  if ch is None:
    ch = 80 if F >= 32 else 200
  if inner is None:
    inner = 25 if F >= 32 else 10
  if nbuf is None:
    nbuf = 5
  nch = _EPW // ch
  outer_n = nch // inner
  blk = inner * ch
  assert inner % nbuf == 0 and nch % inner == 0 and ch % 8 == 0

  @functools.partial(
      pl.kernel,
      out_type=jax.ShapeDtypeStruct((_NC, _N, F), jnp.float32),
      mesh=_sc_mesh(),
      scratch_types=[
          pltpu.VMEM((2, blk), jnp.int32),
          pltpu.VMEM((2, blk), jnp.int32),
          pltpu.VMEM((nbuf, ch, F), jnp.float32),
          pltpu.VMEM_SHARED((_N, F), jnp.float32),
          pltpu.SemaphoreType.DMA,
          pltpu.SemaphoreType.DMA,
          pltpu.SemaphoreType.DMA,
      ],
      compiler_params=pltpu.CompilerParams(use_tc_tiling_on_sc=False),
  )
  def k(g_h, ei_h, zeros_h, out_h, sidx, didx, rows, acc, gsem, ssem, isem):
    c = lax.axis_index("c")
    s = lax.axis_index("s")
    wid = c * _NS + s
    base = wid * _EPW

    @pl.when(s < _NS - 1)
    def _():
      pltpu.sync_copy(zeros_h, acc.at[pl.ds(s * _RPT, _RPT)])

    @pl.when(s == _NS - 1)
    def _():
      pltpu.sync_copy(zeros_h.at[pl.ds(0, _RPT_LAST)],
                      acc.at[pl.ds((_NS - 1) * _RPT, _RPT_LAST)])

    plsc.subcore_barrier()

    drow = zeros_h.at[pl.ds(0, ch)]
    dsrc = ei_h.at[0, pl.ds(0, blk)]

    pltpu.sync_copy(ei_h.at[0, pl.ds(base, blk)], sidx.at[0])
    pltpu.sync_copy(ei_h.at[1, pl.ds(base, blk)], didx.at[0])
    for i in range(nbuf):
      pltpu.async_copy(g_h.at[sidx.at[0, pl.ds(i * ch, ch)]], rows.at[i],
                       gsem)

    def outer(o, carry):
      slot = lax.rem(o, 2)
      nslot = lax.rem(o + 1, 2)
      more = o + 1 < outer_n

      @pl.when(more)
      def _():
        nb = base + (o + 1) * blk
        pltpu.async_copy(ei_h.at[0, pl.ds(nb, blk)], sidx.at[nslot], isem)
        pltpu.async_copy(ei_h.at[1, pl.ds(nb, blk)], didx.at[nslot], isem)

      for i in range(inner):
        b = i % nbuf
        pltpu.make_async_copy(drow, rows.at[b], gsem).wait()
        pltpu.async_copy(rows.at[b],
                         acc.at[didx.at[slot, pl.ds(i * ch, ch)]], ssem,
                         add=True)
        if i == inner - nbuf:
          @pl.when(more)
          def _():
            pltpu.make_async_copy(dsrc, sidx.at[0], isem).wait()
            pltpu.make_async_copy(dsrc, didx.at[0], isem).wait()
        ahead = i + nbuf
        if ahead < inner:
          pltpu.make_async_copy(drow, rows.at[b], ssem).wait()
          pltpu.async_copy(g_h.at[sidx.at[slot, pl.ds(ahead * ch, ch)]],
                           rows.at[b], gsem)
        else:
          @pl.when(more)
          def _():
            pltpu.make_async_copy(drow, rows.at[b], ssem).wait()
            pltpu.async_copy(
                g_h.at[sidx.at[nslot, pl.ds((ahead - inner) * ch, ch)]],
                rows.at[b], gsem)
      return carry

    lax.fori_loop(0, outer_n, outer, 0)
    for _ in range(nbuf):
      pltpu.make_async_copy(drow, rows.at[0], ssem).wait()
    plsc.subcore_barrier()

    @pl.when(s < _NS - 1)
    def _():
      pltpu.sync_copy(acc.at[pl.ds(s * _RPT, _RPT)],
                      out_h.at[c, pl.ds(s * _RPT, _RPT)])

    @pl.when(s == _NS - 1)
    def _():
      pltpu.sync_copy(acc.at[pl.ds((_NS - 1) * _RPT, _RPT_LAST)],
                      out_h.at[c, pl.ds((_NS - 1) * _RPT, _RPT_LAST)])

  zeros = jnp.zeros((_RPT, F), jnp.float32)
  return k(g, ei, zeros)


def _tc_matmul(x, W1):
  def body(x_r, w_r, h_r):
    h_r[...] = jnp.dot(x_r[...], w_r[...],
                       preferred_element_type=jnp.float32)

  F = W1.shape[1]
  return pl.pallas_call(
      body,
      grid=(_G,),
      in_specs=[
          pl.BlockSpec((_BLK, _D), lambda i: (i, 0)),
          pl.BlockSpec((_D, F), lambda i: (0, 0)),
      ],
      out_specs=pl.BlockSpec((_BLK, F), lambda i: (i, 0)),
      out_shape=jax.ShapeDtypeStruct((_N, F), jnp.float32),
  )(x, W1)


def _tc_prep(degp, h1):
  def body(dp_r, h_r, dinv_r, g_r):
    deg = 1.0 + dp_r[0, :, 0:1] + dp_r[1, :, 0:1]
    dinv = lax.rsqrt(deg)
    dinv_r[...] = dinv
    g_r[...] = h_r[...] * dinv

  F = h1.shape[1]
  W = degp.shape[2]
  return pl.pallas_call(
      body,
      grid=(_G,),
      in_specs=[
          pl.BlockSpec((2, _BLK, W), lambda i: (0, i, 0)),
          pl.BlockSpec((_BLK, F), lambda i: (i, 0)),
      ],
      out_specs=[
          pl.BlockSpec((_BLK, 1), lambda i: (i, 0)),
          pl.BlockSpec((_BLK, F), lambda i: (i, 0)),
      ],
      out_shape=[
          jax.ShapeDtypeStruct((_N, 1), jnp.float32),
          jax.ShapeDtypeStruct((_N, F), jnp.float32),
      ],
  )(degp, h1)


def _tc_mid(p, hlin, dinv, b, Wn):
  def body(p_r, h_r, di_r, b_r, w_r, h2_r, g2_r):
    dinv = di_r[...]
    h = jnp.maximum(
        dinv * (p_r[0] + p_r[1]) + (dinv * dinv) * h_r[...] + b_r[...],
        0.0)
    h2 = jnp.dot(h, w_r[...], preferred_element_type=jnp.float32)
    h2_r[...] = h2
    g2_r[...] = h2 * dinv

  Fin, Fout = Wn.shape
  return pl.pallas_call(
      body,
      grid=(_G,),
      in_specs=[
          pl.BlockSpec((2, _BLK, Fin), lambda i: (0, i, 0)),
          pl.BlockSpec((_BLK, Fin), lambda i: (i, 0)),
          pl.BlockSpec((_BLK, 1), lambda i: (i, 0)),
          pl.BlockSpec((1, Fin), lambda i: (0, 0)),
          pl.BlockSpec((Fin, Fout), lambda i: (0, 0)),
      ],
      out_specs=[
          pl.BlockSpec((_BLK, Fout), lambda i: (i, 0)),
          pl.BlockSpec((_BLK, Fout), lambda i: (i, 0)),
      ],
      out_shape=[
          jax.ShapeDtypeStruct((_N, Fout), jnp.float32),
          jax.ShapeDtypeStruct((_N, Fout), jnp.float32),
      ],
  )(p, hlin, dinv, b, Wn)


def _tc_final(p, hlin, dinv, b, Wf, bf):
  def body(p_r, h_r, di_r, b_r, wf_r, bf_r, o_r):
    dinv = di_r[...]
    h = jnp.maximum(
        dinv * (p_r[0] + p_r[1]) + (dinv * dinv) * h_r[...] + b_r[...],
        0.0)
    logits = jnp.dot(h, wf_r[...], preferred_element_type=jnp.float32)
    logits = logits + bf_r[...]
    m = jnp.max(logits, axis=1, keepdims=True)
    ex = jnp.exp(logits - m)
    o_r[...] = logits - m - jnp.log(jnp.sum(ex, axis=1, keepdims=True))

  Fin, Fout = Wf.shape
  return pl.pallas_call(
      body,
      grid=(_G,),
      in_specs=[
          pl.BlockSpec((2, _BLK, Fin), lambda i: (0, i, 0)),
          pl.BlockSpec((_BLK, Fin), lambda i: (i, 0)),
          pl.BlockSpec((_BLK, 1), lambda i: (i, 0)),
          pl.BlockSpec((1, Fin), lambda i: (0, 0)),
          pl.BlockSpec((Fin, Fout), lambda i: (0, 0)),
          pl.BlockSpec((1, Fout), lambda i: (0, 0)),
      ],
      out_specs=pl.BlockSpec((_BLK, Fout), lambda i: (i, 0)),
      out_shape=jax.ShapeDtypeStruct((_N, Fout), jnp.float32),
  )(p, hlin, dinv, b, Wf, bf)


def kernel(x, edge_index, W1, b1, W2, b2, W3, b3, Wf, bf):
  W3p = jnp.pad(W3, ((0, 0), (0, 4)))
  b3p = jnp.pad(b3, (0, 4))
  Wfp = jnp.pad(Wf, ((0, 4), (0, 0)))

  degp = _sc_degree(edge_index)
  h1 = _tc_matmul(x, W1)
  dinv, g1 = _tc_prep(degp, h1)
  p = _sc_scatter(g1, edge_index, W1.shape[1])
  h2, g2 = _tc_mid(p, h1, dinv, b1.reshape(1, -1), W2)
  p = _sc_scatter(g2, edge_index, W2.shape[1])
  h3, g3 = _tc_mid(p, h2, dinv, b2.reshape(1, -1), W3p)
  p = _sc_scatter(g3, edge_index, 16)
  return _tc_final(p, h3, dinv, b3p.reshape(1, -1), Wfp,
                   bf.reshape(1, -1))
